# Initial kernel scaffold; baseline (speedup 1.0000x reference)
import jax
import jax.numpy as jnp
from jax.experimental import pallas as pl


def kernel(x, qkv_w, qkv_b, proj_w, proj_b):
    raise NotImplementedError("write your pallas kernel here")


# speedup vs baseline: 9.4543x; 9.4543x over previous
import jax
import jax.numpy as jnp
from jax.experimental import pallas as pl

_H = 8


def _bilevel_kernel(x_ref, wqkv_ref, bqkv_ref, wproj_ref, bproj_ref,
                    mask_ref, o_ref):
    C = x_ref.shape[-1]
    x = x_ref[0]
    qkv = jnp.dot(x, wqkv_ref[...], preferred_element_type=jnp.float32)
    qkv = qkv + bqkv_ref[...]
    spk = jnp.where(qkv * 0.5 - 1.0 >= 0.0, 1.0, 0.0)
    q = spk[:, :C]
    k16 = spk[:16, C:2 * C]
    v16 = spk[:16, 2 * C:]
    mask = mask_ref[...]
    kv = jnp.dot(k16.T, v16, preferred_element_type=jnp.float32) * mask
    ksum = jnp.sum(k16, axis=0, keepdims=True)
    num = jnp.dot(q, kv, preferred_element_type=jnp.float32)
    den = jnp.dot(q * ksum, mask, preferred_element_type=jnp.float32)
    out = num / (den + 1e-6)
    out = jnp.dot(out, wproj_ref[...], preferred_element_type=jnp.float32)
    o_ref[0] = out + bproj_ref[...]


def kernel(x, qkv_w, qkv_b, proj_w, proj_b):
    T, B, Lt, Lh, Lw, C = x.shape
    rows = Lt * Lh * Lw
    dh = C // _H
    x3 = x.reshape(T * B, rows, C)
    wqkv = qkv_w.T
    bqkv = qkv_b.reshape(1, 3 * C)
    wproj = proj_w.T
    bproj = proj_b.reshape(1, C)
    heads = jnp.arange(C, dtype=jnp.int32) // dh
    mask = (heads[:, None] == heads[None, :]).astype(jnp.float32)

    out = pl.pallas_call(
        _bilevel_kernel,
        grid=(T * B,),
        in_specs=[
            pl.BlockSpec((1, rows, C), lambda i: (i, 0, 0)),
            pl.BlockSpec((C, 3 * C), lambda i: (0, 0)),
            pl.BlockSpec((1, 3 * C), lambda i: (0, 0)),
            pl.BlockSpec((C, C), lambda i: (0, 0)),
            pl.BlockSpec((1, C), lambda i: (0, 0)),
            pl.BlockSpec((C, C), lambda i: (0, 0)),
        ],
        out_specs=pl.BlockSpec((1, rows, C), lambda i: (i, 0, 0)),
        out_shape=jax.ShapeDtypeStruct((T * B, rows, C), jnp.float32),
    )(x3, wqkv, bqkv, wproj, bproj, mask)
    return out.reshape(T, B, Lt, Lh, Lw, C)


# speedup vs baseline: 9.7768x; 1.0341x over previous
import jax
import jax.numpy as jnp
from jax.experimental import pallas as pl

_H = 8


def _bilevel_kernel(x_ref, wq_ref, bq_ref, wkv_ref, bkv_ref, wproj_ref,
                    bproj_ref, mask_ref, o_ref):
    C = x_ref.shape[-1]
    x = x_ref[0]
    qpre = jnp.dot(x, wq_ref[...], preferred_element_type=jnp.float32)
    q = jnp.where(qpre + bq_ref[...] >= 2.0, 1.0, 0.0)
    kvpre = jnp.dot(x[:16], wkv_ref[...], preferred_element_type=jnp.float32)
    skv = jnp.where(kvpre + bkv_ref[...] >= 2.0, 1.0, 0.0)
    k16 = skv[:, :C]
    v16 = skv[:, C:]
    mask = mask_ref[...]
    kv = jnp.dot(k16.T, v16, preferred_element_type=jnp.float32) * mask
    ksum = jnp.sum(k16, axis=0, keepdims=True)
    a = jnp.concatenate([kv, ksum.T * mask], axis=1)
    numden = jnp.dot(q, a, preferred_element_type=jnp.float32)
    out = numden[:, :C] / (numden[:, C:] + 1e-6)
    out = jnp.dot(out, wproj_ref[...], preferred_element_type=jnp.float32)
    o_ref[0] = out + bproj_ref[...]


def kernel(x, qkv_w, qkv_b, proj_w, proj_b):
    T, B, Lt, Lh, Lw, C = x.shape
    rows = Lt * Lh * Lw
    dh = C // _H
    x3 = x.reshape(T * B, rows, C)
    wq = qkv_w[:C].T
    bq = qkv_b[:C].reshape(1, C)
    wkv = qkv_w[C:].T
    bkv = qkv_b[C:].reshape(1, 2 * C)
    wproj = proj_w.T
    bproj = proj_b.reshape(1, C)
    heads = jnp.arange(C, dtype=jnp.int32) // dh
    mask = (heads[:, None] == heads[None, :]).astype(jnp.float32)

    out = pl.pallas_call(
        _bilevel_kernel,
        grid=(T * B,),
        in_specs=[
            pl.BlockSpec((1, rows, C), lambda i: (i, 0, 0)),
            pl.BlockSpec((C, C), lambda i: (0, 0)),
            pl.BlockSpec((1, C), lambda i: (0, 0)),
            pl.BlockSpec((C, 2 * C), lambda i: (0, 0)),
            pl.BlockSpec((1, 2 * C), lambda i: (0, 0)),
            pl.BlockSpec((C, C), lambda i: (0, 0)),
            pl.BlockSpec((1, C), lambda i: (0, 0)),
            pl.BlockSpec((C, C), lambda i: (0, 0)),
        ],
        out_specs=pl.BlockSpec((1, rows, C), lambda i: (i, 0, 0)),
        out_shape=jax.ShapeDtypeStruct((T * B, rows, C), jnp.float32),
    )(x3, wq, bq, wkv, bkv, wproj, bproj, mask)
    return out.reshape(T, B, Lt, Lh, Lw, C)


# speedup vs baseline: 23.2586x; 2.3790x over previous
import jax
import jax.numpy as jnp
from jax.experimental import pallas as pl

_H = 8
_G = 4


def _bilevel_kernel(x_ref, wq_ref, bq_ref, wkv_ref, bkv_ref, wproj_ref,
                    bproj_ref, mask_ref, o_ref):
    C = x_ref.shape[-1]
    rows = x_ref.shape[1]
    x = x_ref[...].reshape(_G * rows, C)
    qpre = jnp.dot(x, wq_ref[...], preferred_element_type=jnp.float32)
    q = jnp.where(qpre + bq_ref[...] >= 2.0, 1.0, 0.0).astype(jnp.bfloat16)
    x16 = jnp.concatenate([x_ref[g, :16] for g in range(_G)], axis=0)
    kvpre = jnp.dot(x16, wkv_ref[...], preferred_element_type=jnp.float32)
    skv = jnp.where(kvpre + bkv_ref[...] >= 2.0, 1.0, 0.0).astype(jnp.bfloat16)
    mask = mask_ref[...]
    outs = []
    for g in range(_G):
        k16 = skv[16 * g:16 * (g + 1), :C]
        v16 = skv[16 * g:16 * (g + 1), C:]
        kv = jnp.dot(k16.T, v16, preferred_element_type=jnp.float32)
        ksum = jnp.sum(k16.astype(jnp.float32), axis=0, keepdims=True)
        a = jnp.concatenate([kv * mask, ksum.T * mask], axis=1)
        numden = jnp.dot(q[rows * g:rows * (g + 1)], a.astype(jnp.bfloat16),
                         preferred_element_type=jnp.float32)
        outs.append(numden[:, :C] / (numden[:, C:] + 1e-6))
    attn = jnp.concatenate(outs, axis=0)
    out = jnp.dot(attn, wproj_ref[...], preferred_element_type=jnp.float32)
    o_ref[...] = (out + bproj_ref[...]).reshape(_G, rows, C)


def kernel(x, qkv_w, qkv_b, proj_w, proj_b):
    T, B, Lt, Lh, Lw, C = x.shape
    rows = Lt * Lh * Lw
    dh = C // _H
    x3 = x.reshape(T * B, rows, C)
    wq = qkv_w[:C].T
    bq = qkv_b[:C].reshape(1, C)
    wkv = qkv_w[C:].T
    bkv = qkv_b[C:].reshape(1, 2 * C)
    wproj = proj_w.T
    bproj = proj_b.reshape(1, C)
    heads = jnp.arange(C, dtype=jnp.int32) // dh
    mask = (heads[:, None] == heads[None, :]).astype(jnp.float32)

    out = pl.pallas_call(
        _bilevel_kernel,
        grid=(T * B // _G,),
        in_specs=[
            pl.BlockSpec((_G, rows, C), lambda i: (i, 0, 0)),
            pl.BlockSpec((C, C), lambda i: (0, 0)),
            pl.BlockSpec((1, C), lambda i: (0, 0)),
            pl.BlockSpec((C, 2 * C), lambda i: (0, 0)),
            pl.BlockSpec((1, 2 * C), lambda i: (0, 0)),
            pl.BlockSpec((C, C), lambda i: (0, 0)),
            pl.BlockSpec((1, C), lambda i: (0, 0)),
            pl.BlockSpec((C, C), lambda i: (0, 0)),
        ],
        out_specs=pl.BlockSpec((_G, rows, C), lambda i: (i, 0, 0)),
        out_shape=jax.ShapeDtypeStruct((T * B, rows, C), jnp.float32),
    )(x3, wq, bq, wkv, bkv, wproj, bproj, mask)
    return out.reshape(T, B, Lt, Lh, Lw, C)


# speedup vs baseline: 25.0102x; 1.0753x over previous
import jax
import jax.numpy as jnp
from jax.experimental import pallas as pl

_H = 8
_G = 4


def _bilevel_kernel(x_ref, wq_ref, bq_ref, wkv_ref, bkv_ref, wproj_ref,
                    bproj_ref, mask_ref, o_ref):
    C = x_ref.shape[-1]
    rows = x_ref.shape[1]
    x = x_ref[...].reshape(_G * rows, C)
    qpre = jnp.dot(x, wq_ref[...], preferred_element_type=jnp.float32)
    q = jnp.where(qpre + bq_ref[...] >= 2.0, 1.0, 0.0).astype(jnp.bfloat16)
    x16 = jnp.concatenate([x_ref[g, :16] for g in range(_G)], axis=0)
    kvpre = jnp.dot(x16, wkv_ref[...], preferred_element_type=jnp.float32)
    skv = jnp.where(kvpre + bkv_ref[...] >= 2.0, 1.0, 0.0).astype(jnp.bfloat16)
    mask = mask_ref[...]
    outs = []
    for g in range(_G):
        k16 = skv[16 * g:16 * (g + 1), :C]
        v16 = skv[16 * g:16 * (g + 1), C:]
        kv = jnp.dot(k16.T, v16,
                     preferred_element_type=jnp.float32
                     ).astype(jnp.bfloat16) * mask
        ksum = jnp.sum(k16.astype(jnp.float32), axis=0,
                       keepdims=True).astype(jnp.bfloat16)
        qg = q[rows * g:rows * (g + 1)]
        num = jnp.dot(qg, kv, preferred_element_type=jnp.float32)
        den = jnp.dot(qg * ksum, mask, preferred_element_type=jnp.float32)
        outs.append(num / (den + 1e-6))
    attn = jnp.concatenate(outs, axis=0)
    out = jnp.dot(attn, wproj_ref[...], preferred_element_type=jnp.float32)
    o_ref[...] = (out + bproj_ref[...]).reshape(_G, rows, C)


def kernel(x, qkv_w, qkv_b, proj_w, proj_b):
    T, B, Lt, Lh, Lw, C = x.shape
    rows = Lt * Lh * Lw
    dh = C // _H
    x3 = x.reshape(T * B, rows, C)
    wq = qkv_w[:C].T
    bq = qkv_b[:C].reshape(1, C)
    wkv = qkv_w[C:].T
    bkv = qkv_b[C:].reshape(1, 2 * C)
    wproj = proj_w.T
    bproj = proj_b.reshape(1, C)
    heads = jnp.arange(C, dtype=jnp.int32) // dh
    mask = (heads[:, None] == heads[None, :]).astype(jnp.bfloat16)

    out = pl.pallas_call(
        _bilevel_kernel,
        grid=(T * B // _G,),
        in_specs=[
            pl.BlockSpec((_G, rows, C), lambda i: (i, 0, 0)),
            pl.BlockSpec((C, C), lambda i: (0, 0)),
            pl.BlockSpec((1, C), lambda i: (0, 0)),
            pl.BlockSpec((C, 2 * C), lambda i: (0, 0)),
            pl.BlockSpec((1, 2 * C), lambda i: (0, 0)),
            pl.BlockSpec((C, C), lambda i: (0, 0)),
            pl.BlockSpec((1, C), lambda i: (0, 0)),
            pl.BlockSpec((C, C), lambda i: (0, 0)),
        ],
        out_specs=pl.BlockSpec((_G, rows, C), lambda i: (i, 0, 0)),
        out_shape=jax.ShapeDtypeStruct((T * B, rows, C), jnp.float32),
    )(x3, wq, bq, wkv, bkv, wproj, bproj, mask)
    return out.reshape(T, B, Lt, Lh, Lw, C)


# speedup vs baseline: 31.1600x; 1.2459x over previous
import jax
import jax.numpy as jnp
from jax.experimental import pallas as pl

_H = 8
_G = 8


def _bilevel_kernel(x_ref, wq_ref, bq_ref, wkv_ref, bkv_ref, wproj_ref,
                    bproj_ref, mask_ref, o_ref):
    C = x_ref.shape[-1]
    rows = x_ref.shape[1]
    x = x_ref[...].reshape(_G * rows, C)
    qpre = jnp.dot(x, wq_ref[...], preferred_element_type=jnp.float32)
    q = jnp.where(qpre + bq_ref[...] >= 2.0, 1.0, 0.0).astype(jnp.bfloat16)
    x16 = jnp.concatenate([x_ref[g, :16] for g in range(_G)], axis=0)
    kvpre = jnp.dot(x16, wkv_ref[...], preferred_element_type=jnp.float32)
    skv = jnp.where(kvpre + bkv_ref[...] >= 2.0, 1.0, 0.0).astype(jnp.bfloat16)
    mask = mask_ref[...]
    outs = []
    for g in range(_G):
        k16 = skv[16 * g:16 * (g + 1), :C]
        v16 = skv[16 * g:16 * (g + 1), C:]
        kv = jnp.dot(k16.T, v16,
                     preferred_element_type=jnp.float32
                     ).astype(jnp.bfloat16) * mask
        ksum = jnp.sum(k16.astype(jnp.float32), axis=0,
                       keepdims=True).astype(jnp.bfloat16)
        qg = q[rows * g:rows * (g + 1)]
        num = jnp.dot(qg, kv, preferred_element_type=jnp.float32)
        den = jnp.dot(qg * ksum, mask, preferred_element_type=jnp.float32)
        outs.append(num / (den + 1e-6))
    attn = jnp.concatenate(outs, axis=0)
    out = jnp.dot(attn, wproj_ref[...], preferred_element_type=jnp.float32)
    o_ref[...] = (out + bproj_ref[...]).reshape(_G, rows, C)


def kernel(x, qkv_w, qkv_b, proj_w, proj_b):
    T, B, Lt, Lh, Lw, C = x.shape
    rows = Lt * Lh * Lw
    dh = C // _H
    x3 = x.reshape(T * B, rows, C)
    wq = qkv_w[:C].T
    bq = qkv_b[:C].reshape(1, C)
    wkv = qkv_w[C:].T
    bkv = qkv_b[C:].reshape(1, 2 * C)
    wproj = proj_w.T
    bproj = proj_b.reshape(1, C)
    heads = jnp.arange(C, dtype=jnp.int32) // dh
    mask = (heads[:, None] == heads[None, :]).astype(jnp.bfloat16)

    out = pl.pallas_call(
        _bilevel_kernel,
        grid=(T * B // _G,),
        in_specs=[
            pl.BlockSpec((_G, rows, C), lambda i: (i, 0, 0)),
            pl.BlockSpec((C, C), lambda i: (0, 0)),
            pl.BlockSpec((1, C), lambda i: (0, 0)),
            pl.BlockSpec((C, 2 * C), lambda i: (0, 0)),
            pl.BlockSpec((1, 2 * C), lambda i: (0, 0)),
            pl.BlockSpec((C, C), lambda i: (0, 0)),
            pl.BlockSpec((1, C), lambda i: (0, 0)),
            pl.BlockSpec((C, C), lambda i: (0, 0)),
        ],
        out_specs=pl.BlockSpec((_G, rows, C), lambda i: (i, 0, 0)),
        out_shape=jax.ShapeDtypeStruct((T * B, rows, C), jnp.float32),
    )(x3, wq, bq, wkv, bkv, wproj, bproj, mask)
    return out.reshape(T, B, Lt, Lh, Lw, C)


# speedup vs baseline: 39.9797x; 1.2830x over previous
import jax
import jax.numpy as jnp
from jax.experimental import pallas as pl

_H = 8
_G = 16


def _bilevel_kernel(x_ref, wq_ref, bq_ref, wkv_ref, bkv_ref, wproj_ref,
                    bproj_ref, mask_ref, o_ref):
    C = x_ref.shape[-1]
    rows = x_ref.shape[1]
    x = x_ref[...].reshape(_G * rows, C)
    qpre = jnp.dot(x, wq_ref[...], preferred_element_type=jnp.float32)
    q = jnp.where(qpre + bq_ref[...] >= 2.0, 1.0, 0.0).astype(jnp.bfloat16)
    x16 = jnp.concatenate([x_ref[g, :16] for g in range(_G)], axis=0)
    kvpre = jnp.dot(x16, wkv_ref[...], preferred_element_type=jnp.float32)
    skv = jnp.where(kvpre + bkv_ref[...] >= 2.0, 1.0, 0.0).astype(jnp.bfloat16)
    mask = mask_ref[...]
    outs = []
    for g in range(_G):
        k16 = skv[16 * g:16 * (g + 1), :C]
        v16 = skv[16 * g:16 * (g + 1), C:]
        kv = jnp.dot(k16.T, v16,
                     preferred_element_type=jnp.float32
                     ).astype(jnp.bfloat16) * mask
        ksum = jnp.sum(k16.astype(jnp.float32), axis=0,
                       keepdims=True).astype(jnp.bfloat16)
        qg = q[rows * g:rows * (g + 1)]
        num = jnp.dot(qg, kv, preferred_element_type=jnp.float32)
        den = jnp.dot(qg * ksum, mask, preferred_element_type=jnp.float32)
        outs.append(num / (den + 1e-6))
    attn = jnp.concatenate(outs, axis=0)
    out = jnp.dot(attn, wproj_ref[...], preferred_element_type=jnp.float32)
    o_ref[...] = (out + bproj_ref[...]).reshape(_G, rows, C)


def kernel(x, qkv_w, qkv_b, proj_w, proj_b):
    T, B, Lt, Lh, Lw, C = x.shape
    rows = Lt * Lh * Lw
    dh = C // _H
    x3 = x.reshape(T * B, rows, C)
    wq = qkv_w[:C].T
    bq = qkv_b[:C].reshape(1, C)
    wkv = qkv_w[C:].T
    bkv = qkv_b[C:].reshape(1, 2 * C)
    wproj = proj_w.T
    bproj = proj_b.reshape(1, C)
    heads = jnp.arange(C, dtype=jnp.int32) // dh
    mask = (heads[:, None] == heads[None, :]).astype(jnp.bfloat16)

    out = pl.pallas_call(
        _bilevel_kernel,
        grid=(T * B // _G,),
        in_specs=[
            pl.BlockSpec((_G, rows, C), lambda i: (i, 0, 0)),
            pl.BlockSpec((C, C), lambda i: (0, 0)),
            pl.BlockSpec((1, C), lambda i: (0, 0)),
            pl.BlockSpec((C, 2 * C), lambda i: (0, 0)),
            pl.BlockSpec((1, 2 * C), lambda i: (0, 0)),
            pl.BlockSpec((C, C), lambda i: (0, 0)),
            pl.BlockSpec((1, C), lambda i: (0, 0)),
            pl.BlockSpec((C, C), lambda i: (0, 0)),
        ],
        out_specs=pl.BlockSpec((_G, rows, C), lambda i: (i, 0, 0)),
        out_shape=jax.ShapeDtypeStruct((T * B, rows, C), jnp.float32),
    )(x3, wq, bq, wkv, bkv, wproj, bproj, mask)
    return out.reshape(T, B, Lt, Lh, Lw, C)


# speedup vs baseline: 41.8018x; 1.0456x over previous
import jax
import jax.numpy as jnp
from jax.experimental import pallas as pl

_H = 8
_G = 32


def _bilevel_kernel(x_ref, wq_ref, bq_ref, wkv_ref, bkv_ref, wproj_ref,
                    bproj_ref, mask_ref, o_ref):
    C = x_ref.shape[-1]
    rows = x_ref.shape[1]
    x = x_ref[...].reshape(_G * rows, C)
    qpre = jnp.dot(x, wq_ref[...], preferred_element_type=jnp.float32)
    q = jnp.where(qpre + bq_ref[...] >= 2.0, 1.0, 0.0).astype(jnp.bfloat16)
    x16 = jnp.concatenate([x_ref[g, :16] for g in range(_G)], axis=0)
    kvpre = jnp.dot(x16, wkv_ref[...], preferred_element_type=jnp.float32)
    skv = jnp.where(kvpre + bkv_ref[...] >= 2.0, 1.0, 0.0).astype(jnp.bfloat16)
    mask = mask_ref[...]
    outs = []
    for g in range(_G):
        k16 = skv[16 * g:16 * (g + 1), :C]
        v16 = skv[16 * g:16 * (g + 1), C:]
        kv = jnp.dot(k16.T, v16,
                     preferred_element_type=jnp.float32
                     ).astype(jnp.bfloat16) * mask
        ksum = jnp.sum(k16.astype(jnp.float32), axis=0,
                       keepdims=True).astype(jnp.bfloat16)
        qg = q[rows * g:rows * (g + 1)]
        num = jnp.dot(qg, kv, preferred_element_type=jnp.float32)
        den = jnp.dot(qg * ksum, mask, preferred_element_type=jnp.float32)
        outs.append(num / (den + 1e-6))
    attn = jnp.concatenate(outs, axis=0)
    out = jnp.dot(attn, wproj_ref[...], preferred_element_type=jnp.float32)
    o_ref[...] = (out + bproj_ref[...]).reshape(_G, rows, C)


def kernel(x, qkv_w, qkv_b, proj_w, proj_b):
    T, B, Lt, Lh, Lw, C = x.shape
    rows = Lt * Lh * Lw
    dh = C // _H
    x3 = x.reshape(T * B, rows, C)
    wq = qkv_w[:C].T
    bq = qkv_b[:C].reshape(1, C)
    wkv = qkv_w[C:].T
    bkv = qkv_b[C:].reshape(1, 2 * C)
    wproj = proj_w.T
    bproj = proj_b.reshape(1, C)
    heads = jnp.arange(C, dtype=jnp.int32) // dh
    mask = (heads[:, None] == heads[None, :]).astype(jnp.bfloat16)

    out = pl.pallas_call(
        _bilevel_kernel,
        grid=(T * B // _G,),
        in_specs=[
            pl.BlockSpec((_G, rows, C), lambda i: (i, 0, 0)),
            pl.BlockSpec((C, C), lambda i: (0, 0)),
            pl.BlockSpec((1, C), lambda i: (0, 0)),
            pl.BlockSpec((C, 2 * C), lambda i: (0, 0)),
            pl.BlockSpec((1, 2 * C), lambda i: (0, 0)),
            pl.BlockSpec((C, C), lambda i: (0, 0)),
            pl.BlockSpec((1, C), lambda i: (0, 0)),
            pl.BlockSpec((C, C), lambda i: (0, 0)),
        ],
        out_specs=pl.BlockSpec((_G, rows, C), lambda i: (i, 0, 0)),
        out_shape=jax.ShapeDtypeStruct((T * B, rows, C), jnp.float32),
    )(x3, wq, bq, wkv, bkv, wproj, bproj, mask)
    return out.reshape(T, B, Lt, Lh, Lw, C)
